# baseline (speedup 1.0000x reference)
import functools

import jax
import jax.numpy as jnp
from jax import lax
from jax.experimental import pallas as pl
from jax.experimental.pallas import tpu as pltpu
from jax.experimental.pallas import tpu_sc as plsc


def kernel(vertices, joints, extra_joints_idxs):
    del vertices, extra_joints_idxs
    n, j, c = joints.shape
    t = joints.transpose(2, 1, 0)
    info = plsc.get_sparse_core_info()
    nw = info.num_cores * info.num_subcores
    w = n // nw
    mesh = plsc.VectorSubcoreMesh(core_axis_name="c", subcore_axis_name="s")

    @functools.partial(
        pl.kernel,
        mesh=mesh,
        out_type=jax.ShapeDtypeStruct((c, j, n), joints.dtype),
        scratch_types=[
            pltpu.VMEM((c, j, w), joints.dtype),
            pltpu.SemaphoreType.DMA,
        ],
    )
    def k(src, out, buf, sem):
        wid = lax.axis_index("s") * info.num_cores + lax.axis_index("c")
        sl = pl.ds(wid * w, w)
        pltpu.async_copy(src.at[:, :, sl], buf, sem).wait()
        pltpu.sync_copy(buf, out.at[:, :, sl])

    return k(t).transpose(2, 1, 0)


# speedup vs baseline: 7.2251x; 7.2251x over previous
import jax
import jax.numpy as jnp
from jax.experimental import pallas as pl
from jax.experimental.pallas import tpu as pltpu


def _copy_body(j_ref, o_ref, vmem, in_sems, out_sems):
    c, j, n = j_ref.shape
    edges = (0, 512, 2048, n)
    chunks = [(i, slice(edges[q], edges[q + 1]))
              for q in range(len(edges) - 1) for i in range(c)]
    ins = []
    for k, (i, s) in enumerate(chunks):
        cp = pltpu.make_async_copy(j_ref.at[i, :, s], vmem.at[i, :, s], in_sems.at[k])
        cp.start()
        ins.append(cp)
    outs = []
    for k, (i, s) in enumerate(chunks):
        ins[k].wait()
        cp = pltpu.make_async_copy(vmem.at[i, :, s], o_ref.at[i, :, s], out_sems.at[k])
        cp.start()
        outs.append(cp)
    for cp in outs:
        cp.wait()


def kernel(vertices, joints, extra_joints_idxs):
    del vertices, extra_joints_idxs
    n, j, c = joints.shape
    t = joints.transpose(2, 1, 0)
    out_t = pl.pallas_call(
        _copy_body,
        in_specs=[pl.BlockSpec(memory_space=pl.ANY)],
        out_specs=pl.BlockSpec(memory_space=pl.ANY),
        out_shape=jax.ShapeDtypeStruct((c, j, n), joints.dtype),
        scratch_shapes=[
            pltpu.VMEM((c, j, n), joints.dtype),
            pltpu.SemaphoreType.DMA((3 * c,)),
            pltpu.SemaphoreType.DMA((3 * c,)),
        ],
    )(t)
    return out_t.transpose(2, 1, 0)


# speedup vs baseline: 7.3269x; 1.0141x over previous
import jax
import jax.numpy as jnp
from jax.experimental import pallas as pl
from jax.experimental.pallas import tpu as pltpu


def _copy_body(j_ref, o_ref, vmem, in_sems, out_sems):
    c, j, n = j_ref.shape
    edges = (0, 2048, 3584, n)
    chunks = [(i, slice(edges[q], edges[q + 1]))
              for q in range(len(edges) - 1) for i in range(c)]
    ins = []
    for k, (i, s) in enumerate(chunks):
        cp = pltpu.make_async_copy(j_ref.at[i, :, s], vmem.at[i, :, s], in_sems.at[k])
        cp.start()
        ins.append(cp)
    outs = []
    for k, (i, s) in enumerate(chunks):
        ins[k].wait()
        cp = pltpu.make_async_copy(vmem.at[i, :, s], o_ref.at[i, :, s], out_sems.at[k])
        cp.start()
        outs.append(cp)
    for cp in outs:
        cp.wait()


def kernel(vertices, joints, extra_joints_idxs):
    del vertices, extra_joints_idxs
    n, j, c = joints.shape
    t = joints.transpose(2, 1, 0)
    out_t = pl.pallas_call(
        _copy_body,
        in_specs=[pl.BlockSpec(memory_space=pl.ANY)],
        out_specs=pl.BlockSpec(memory_space=pl.ANY),
        out_shape=jax.ShapeDtypeStruct((c, j, n), joints.dtype),
        scratch_shapes=[
            pltpu.VMEM((c, j, n), joints.dtype),
            pltpu.SemaphoreType.DMA((3 * c,)),
            pltpu.SemaphoreType.DMA((3 * c,)),
        ],
    )(t)
    return out_t.transpose(2, 1, 0)


# speedup vs baseline: 7.4659x; 1.0190x over previous
import jax
import jax.numpy as jnp
from jax.experimental import pallas as pl
from jax.experimental.pallas import tpu as pltpu


def _copy_body(j_ref, o_ref, vmem, in_sems, out_sems):
    c, j, n = j_ref.shape
    nsplit = 2
    w = n // nsplit
    chunks = [(i, slice(q * w, (q + 1) * w))
              for i in range(c) for q in range(nsplit)]
    ins = []
    for k, (i, s) in enumerate(chunks):
        cp = pltpu.make_async_copy(j_ref.at[i, :, s], vmem.at[i, :, s], in_sems.at[k])
        cp.start()
        ins.append(cp)
    outs = []
    for k, (i, s) in enumerate(chunks):
        ins[k].wait()
        cp = pltpu.make_async_copy(vmem.at[i, :, s], o_ref.at[i, :, s], out_sems.at[k])
        cp.start()
        outs.append(cp)
    for cp in outs:
        cp.wait()


def kernel(vertices, joints, extra_joints_idxs):
    del vertices, extra_joints_idxs
    n, j, c = joints.shape
    t = joints.transpose(2, 1, 0)
    out_t = pl.pallas_call(
        _copy_body,
        in_specs=[pl.BlockSpec(memory_space=pl.ANY)],
        out_specs=pl.BlockSpec(memory_space=pl.ANY),
        out_shape=jax.ShapeDtypeStruct((c, j, n), joints.dtype),
        scratch_shapes=[
            pltpu.VMEM((c, j, n), joints.dtype),
            pltpu.SemaphoreType.DMA((2 * c,)),
            pltpu.SemaphoreType.DMA((2 * c,)),
        ],
    )(t)
    return out_t.transpose(2, 1, 0)
